# Initial kernel scaffold; baseline (speedup 1.0000x reference)
import jax
import jax.numpy as jnp
from jax.experimental import pallas as pl


def kernel(gates, attention_mask):
    raise NotImplementedError("write your pallas kernel here")


# speedup vs baseline: 77.0959x; 77.0959x over previous
import jax
import jax.numpy as jnp
from jax import lax
from jax.experimental import pallas as pl

_THRESHOLD = 0.9
_N = 32768
_ROWS = 64
_BLOCK_ROWS = 8
_ONE_PATTERN = 0x3F800000


def _keep_mask_body(g_ref, m_ref, o_ref):
    g = g_ref[...]
    act = m_ref[...] != 0
    gated = jnp.where(act, g, 0.0)
    total = jnp.maximum(jnp.sum(gated, axis=-1, keepdims=True), 1e-12)
    s = jnp.where(act, gated / total, 0.0)

    f0 = jnp.sum(s, axis=-1, keepdims=True)
    allkeep = f0 <= _THRESHOLD

    lo0 = jnp.zeros((_BLOCK_ROWS, 1), jnp.int32)
    hi0 = jnp.full((_BLOCK_ROWS, 1), _ONE_PATTERN, jnp.int32)

    def vstep(_, carry):
        lo, hi = carry
        mid = (lo + hi) >> 1
        v = lax.bitcast_convert_type(mid, jnp.float32)
        above = jnp.sum(jnp.where(s > v, s, 0.0), axis=-1, keepdims=True)
        over = above > _THRESHOLD
        return jnp.where(over, mid, lo), jnp.where(over, hi, mid)

    _, hi = lax.fori_loop(0, 30, vstep, (lo0, hi0))
    v = lax.bitcast_convert_type(hi, jnp.float32)

    gt = s > v
    f_hi = jnp.sum(jnp.where(gt, s, 0.0), axis=-1, keepdims=True)
    c_hi = jnp.sum(jnp.where(gt, 1.0, 0.0), axis=-1, keepdims=True)
    eq = (s == v) & act
    idx = lax.broadcasted_iota(jnp.int32, (_BLOCK_ROWS, _N), 1)

    jlo0 = jnp.zeros((_BLOCK_ROWS, 1), jnp.int32)
    jhi0 = jnp.full((_BLOCK_ROWS, 1), _N, jnp.int32)

    def jstep(_, carry):
        jlo, jhi = carry
        jmid = (jlo + jhi) >> 1
        cnt = jnp.sum(jnp.where(eq & (idx < jmid), 1.0, 0.0),
                      axis=-1, keepdims=True)
        ok = f_hi + cnt * v <= _THRESHOLD
        return jnp.where(ok, jmid, jlo), jnp.where(ok, jhi, jmid)

    jlo, _ = lax.fori_loop(0, 15, jstep, (jlo0, jhi0))
    keep_eq = eq & (idx < jlo)

    cnt_kept = jnp.sum(jnp.where(keep_eq, 1.0, 0.0), axis=-1, keepdims=True)
    min_eq_idx = jnp.min(jnp.where(eq, idx, _N), axis=-1, keepdims=True)
    force = (c_hi == 0.0) & (cnt_kept == 0.0)
    keep_eq = keep_eq | (force & (idx == min_eq_idx) & eq)

    keep = ((gt | keep_eq) | allkeep) & act
    o_ref[...] = jnp.where(keep, 1, 0)


def kernel(gates, attention_mask):
    out = pl.pallas_call(
        _keep_mask_body,
        grid=(_ROWS // _BLOCK_ROWS,),
        in_specs=[
            pl.BlockSpec((_BLOCK_ROWS, _N), lambda i: (i, 0)),
            pl.BlockSpec((_BLOCK_ROWS, _N), lambda i: (i, 0)),
        ],
        out_specs=pl.BlockSpec((_BLOCK_ROWS, _N), lambda i: (i, 0)),
        out_shape=jax.ShapeDtypeStruct((_ROWS, _N), jnp.int32),
    )(gates, attention_mask)
    return out.astype(jnp.bool_)


# speedup vs baseline: 104.0530x; 1.3497x over previous
import jax
import jax.numpy as jnp
from jax import lax
from jax.experimental import pallas as pl

_THRESHOLD = 0.9
_N = 32768
_ROWS = 64
_BLOCK_ROWS = 8
_ONE_PATTERN = 0x3F800000


def _keep_mask_body(g_ref, m_ref, o_ref):
    g = g_ref[...]
    act = m_ref[...] != 0
    gated = jnp.where(act, g, 0.0)
    total = jnp.maximum(jnp.sum(gated, axis=-1, keepdims=True), 1e-12)
    s = jnp.where(act, gated / total, 0.0)

    f0 = jnp.sum(s, axis=-1, keepdims=True)
    allkeep = f0 <= _THRESHOLD

    lo0 = jnp.zeros((_BLOCK_ROWS, 1), jnp.int32)
    hi0 = jnp.full((_BLOCK_ROWS, 1), _ONE_PATTERN, jnp.int32)

    def vstep(_, carry):
        lo, hi = carry
        mid = (lo + hi) >> 1
        v = lax.bitcast_convert_type(mid, jnp.float32)
        above = jnp.sum(jnp.where(s > v, s, 0.0), axis=-1, keepdims=True)
        over = above > _THRESHOLD
        return jnp.where(over, mid, lo), jnp.where(over, hi, mid)

    _, hi = lax.fori_loop(0, 30, vstep, (lo0, hi0))
    v = lax.bitcast_convert_type(hi, jnp.float32)

    gt = s > v
    f_hi = jnp.sum(jnp.where(gt, s, 0.0), axis=-1, keepdims=True)
    c_hi = jnp.sum(jnp.where(gt, 1.0, 0.0), axis=-1, keepdims=True)
    eq = (s == v) & act
    idx = lax.broadcasted_iota(jnp.int32, (_BLOCK_ROWS, _N), 1)

    min_eq_idx = jnp.min(jnp.where(eq, idx, _N), axis=-1, keepdims=True)
    max_eq_idx = jnp.max(jnp.where(eq, idx, -1), axis=-1, keepdims=True)
    e_cnt = jnp.sum(jnp.where(eq, 1.0, 0.0), axis=-1, keepdims=True)
    all_fit = f_hi + e_cnt * v <= _THRESHOLD
    jhi0 = max_eq_idx + 1
    jlo0 = jnp.where(all_fit, jhi0, min_eq_idx)

    def jcond(carry):
        jlo, jhi = carry
        return jnp.any(jhi - jlo > 1)

    def jstep(carry):
        jlo, jhi = carry
        jmid = (jlo + jhi) >> 1
        cnt = jnp.sum(jnp.where(eq & (idx < jmid), 1.0, 0.0),
                      axis=-1, keepdims=True)
        ok = f_hi + cnt * v <= _THRESHOLD
        return jnp.where(ok, jmid, jlo), jnp.where(ok, jhi, jmid)

    jlo, _ = lax.while_loop(jcond, jstep, (jlo0, jhi0))
    keep_eq = eq & (idx < jlo)

    cnt_kept = jnp.sum(jnp.where(keep_eq, 1.0, 0.0), axis=-1, keepdims=True)
    min_eq_idx = jnp.min(jnp.where(eq, idx, _N), axis=-1, keepdims=True)
    force = (c_hi == 0.0) & (cnt_kept == 0.0)
    keep_eq = keep_eq | (force & (idx == min_eq_idx) & eq)

    keep = ((gt | keep_eq) | allkeep) & act
    o_ref[...] = jnp.where(keep, 1, 0)


def kernel(gates, attention_mask):
    out = pl.pallas_call(
        _keep_mask_body,
        grid=(_ROWS // _BLOCK_ROWS,),
        in_specs=[
            pl.BlockSpec((_BLOCK_ROWS, _N), lambda i: (i, 0)),
            pl.BlockSpec((_BLOCK_ROWS, _N), lambda i: (i, 0)),
        ],
        out_specs=pl.BlockSpec((_BLOCK_ROWS, _N), lambda i: (i, 0)),
        out_shape=jax.ShapeDtypeStruct((_ROWS, _N), jnp.int32),
    )(gates, attention_mask)
    return out.astype(jnp.bool_)
